# Initial kernel scaffold; baseline (speedup 1.0000x reference)
import jax
import jax.numpy as jnp
from jax.experimental import pallas as pl


def kernel(x, Wr, W1, b1, W2, b2):
    raise NotImplementedError("write your pallas kernel here")


# speedup vs baseline: 1.7091x; 1.7091x over previous
import functools

import jax
import jax.numpy as jnp
from jax.experimental import pallas as pl
from jax.experimental.pallas import tpu as pltpu

E = 16
H = 1024
FFN = 2048
TOP_K = 2
T = 128
F_CHUNK = 512
N_CHUNKS = FFN // F_CHUNK


def _moe_body(x_ref, wr_ref, w1_ref, b1_ref, w2_ref, b2_ref,
              out_ref, probs_ref, topk_ref, c_ref):
    e = pl.program_id(0)
    f = pl.program_id(1)

    @pl.when((e == 0) & (f == 0))
    def _router():
        xx = x_ref[...]
        g = jax.lax.dot_general(
            xx, wr_ref[...],
            dimension_numbers=(((1,), (1,)), ((), ())),
            preferred_element_type=jnp.float32)
        m = jnp.max(g, axis=-1, keepdims=True)
        ex = jnp.exp(g - m)
        p = ex / jnp.sum(ex, axis=-1, keepdims=True)
        probs_ref[...] = p
        cols = jax.lax.broadcasted_iota(jnp.int32, (T, E), 1)
        i1 = jnp.argmax(p, axis=-1)
        p1 = jnp.max(p, axis=-1)
        pm = jnp.where(cols == i1[:, None], -1.0, p)
        i2 = jnp.argmax(pm, axis=-1)
        p2 = jnp.max(pm, axis=-1)
        s = p1 + p2 + 1e-9
        p1n = (p1 / s)[:, None]
        p2n = (p2 / s)[:, None]
        topk_ref[...] = jnp.concatenate([i1[:, None], i2[:, None]], axis=1)
        c_ref[...] = (jnp.where(cols == i1[:, None], p1n, 0.0)
                      + jnp.where(cols == i2[:, None], p2n, 0.0))
        out_ref[...] = jnp.zeros_like(out_ref)

    cols = jax.lax.broadcasted_iota(jnp.int32, (T, E), 1)
    coef = jnp.sum(jnp.where(cols == e, c_ref[...], 0.0), axis=1,
                   keepdims=True)

    xb = x_ref[...].astype(jnp.bfloat16)
    w1 = w1_ref[0].astype(jnp.bfloat16)
    h = jax.lax.dot_general(
        xb, w1, dimension_numbers=(((1,), (1,)), ((), ())),
        preferred_element_type=jnp.float32)
    h = h + b1_ref[0]
    h = 0.5 * h * (1.0 + jax.lax.erf(h * 0.7071067811865476))
    w2 = w2_ref[0].astype(jnp.bfloat16)
    o = jax.lax.dot_general(
        h.astype(jnp.bfloat16), w2,
        dimension_numbers=(((1,), (1,)), ((), ())),
        preferred_element_type=jnp.float32)

    upd = coef * o
    @pl.when(f == 0)
    def _bias():
        out_ref[...] += coef * b2_ref[0]
    out_ref[...] += upd


@jax.jit
def kernel(x, Wr, W1, b1, W2, b2):
    B, S, _ = x.shape
    xf = x.reshape(T, H)

    grid = (E, N_CHUNKS)
    out, probs, topk = pl.pallas_call(
        _moe_body,
        grid=grid,
        in_specs=[
            pl.BlockSpec((T, H), lambda e, f: (0, 0)),
            pl.BlockSpec((E, H), lambda e, f: (0, 0)),
            pl.BlockSpec((1, F_CHUNK, H), lambda e, f: (e, f, 0)),
            pl.BlockSpec((1, 1, F_CHUNK), lambda e, f: (e, 0, f)),
            pl.BlockSpec((1, H, F_CHUNK), lambda e, f: (e, 0, f)),
            pl.BlockSpec((1, 1, H), lambda e, f: (e, 0, 0)),
        ],
        out_specs=[
            pl.BlockSpec((T, H), lambda e, f: (0, 0)),
            pl.BlockSpec((T, E), lambda e, f: (0, 0)),
            pl.BlockSpec((T, TOP_K), lambda e, f: (0, 0)),
        ],
        out_shape=[
            jax.ShapeDtypeStruct((T, H), jnp.float32),
            jax.ShapeDtypeStruct((T, E), jnp.float32),
            jax.ShapeDtypeStruct((T, TOP_K), jnp.int32),
        ],
        scratch_shapes=[pltpu.VMEM((T, E), jnp.float32)],
        compiler_params=pltpu.CompilerParams(
            dimension_semantics=("arbitrary", "arbitrary"),
        ),
    )(xf, Wr, W1, b1.reshape(E, 1, FFN), W2, b2.reshape(E, 1, H))

    return (out.reshape(B, S, H), probs.reshape(B, S, E),
            topk.reshape(B, S, TOP_K))


# speedup vs baseline: 1.7175x; 1.0049x over previous
import functools

import jax
import jax.numpy as jnp
from jax.experimental import pallas as pl
from jax.experimental.pallas import tpu as pltpu

E = 16
H = 1024
FFN = 2048
TOP_K = 2
T = 128
F_CHUNK = 512
N_CHUNKS = FFN // F_CHUNK


def _moe_body(x_ref, wr_ref, w1_ref, b1_ref, w2_ref, b2_ref,
              out_ref, probs_ref, topk_ref, c_ref):
    e = pl.program_id(0)
    f = pl.program_id(1)

    @pl.when((e == 0) & (f == 0))
    def _router():
        xx = x_ref[...]
        g = jax.lax.dot_general(
            xx, wr_ref[...],
            dimension_numbers=(((1,), (1,)), ((), ())),
            preferred_element_type=jnp.float32)
        m = jnp.max(g, axis=-1, keepdims=True)
        ex = jnp.exp(g - m)
        p = ex / jnp.sum(ex, axis=-1, keepdims=True)
        probs_ref[...] = p
        cols = jax.lax.broadcasted_iota(jnp.int32, (T, E), 1)
        i1 = jnp.argmax(p, axis=-1)
        p1 = jnp.max(p, axis=-1)
        pm = jnp.where(cols == i1[:, None], -1.0, p)
        i2 = jnp.argmax(pm, axis=-1)
        p2 = jnp.max(pm, axis=-1)
        s = p1 + p2 + 1e-9
        p1n = (p1 / s)[:, None]
        p2n = (p2 / s)[:, None]
        topk_ref[...] = jnp.concatenate([i1[:, None], i2[:, None]], axis=1)
        c_ref[...] = (jnp.where(cols == i1[:, None], p1n, 0.0)
                      + jnp.where(cols == i2[:, None], p2n, 0.0))
        out_ref[...] = jnp.zeros_like(out_ref)

    cols = jax.lax.broadcasted_iota(jnp.int32, (T, E), 1)
    coef = jnp.sum(jnp.where(cols == e, c_ref[...], 0.0), axis=1,
                   keepdims=True)

    h = jax.lax.dot_general(
        x_ref[...], w1_ref[0], dimension_numbers=(((1,), (1,)), ((), ())),
        preferred_element_type=jnp.float32,
        precision=jax.lax.Precision.DEFAULT)
    h = h + b1_ref[0]
    h = 0.5 * h * (1.0 + jax.lax.erf(h * 0.7071067811865476))
    o = jax.lax.dot_general(
        h, w2_ref[0],
        dimension_numbers=(((1,), (1,)), ((), ())),
        preferred_element_type=jnp.float32,
        precision=jax.lax.Precision.DEFAULT)

    upd = coef * o
    @pl.when(f == 0)
    def _bias():
        out_ref[...] += coef * b2_ref[0]
    out_ref[...] += upd


@jax.jit
def kernel(x, Wr, W1, b1, W2, b2):
    B, S, _ = x.shape
    xf = x.reshape(T, H)

    grid = (E, N_CHUNKS)
    out, probs, topk = pl.pallas_call(
        _moe_body,
        grid=grid,
        in_specs=[
            pl.BlockSpec((T, H), lambda e, f: (0, 0)),
            pl.BlockSpec((E, H), lambda e, f: (0, 0)),
            pl.BlockSpec((1, F_CHUNK, H), lambda e, f: (e, f, 0)),
            pl.BlockSpec((1, 1, F_CHUNK), lambda e, f: (e, 0, f)),
            pl.BlockSpec((1, H, F_CHUNK), lambda e, f: (e, 0, f)),
            pl.BlockSpec((1, 1, H), lambda e, f: (e, 0, 0)),
        ],
        out_specs=[
            pl.BlockSpec((T, H), lambda e, f: (0, 0)),
            pl.BlockSpec((T, E), lambda e, f: (0, 0)),
            pl.BlockSpec((T, TOP_K), lambda e, f: (0, 0)),
        ],
        out_shape=[
            jax.ShapeDtypeStruct((T, H), jnp.float32),
            jax.ShapeDtypeStruct((T, E), jnp.float32),
            jax.ShapeDtypeStruct((T, TOP_K), jnp.int32),
        ],
        scratch_shapes=[pltpu.VMEM((T, E), jnp.float32)],
        compiler_params=pltpu.CompilerParams(
            dimension_semantics=("arbitrary", "arbitrary"),
        ),
    )(xf, Wr, W1, b1.reshape(E, 1, FFN), W2, b2.reshape(E, 1, H))

    return (out.reshape(B, S, H), probs.reshape(B, S, E),
            topk.reshape(B, S, TOP_K))


# speedup vs baseline: 2.0642x; 1.2019x over previous
import jax
import jax.numpy as jnp
from jax.experimental import pallas as pl
from jax.experimental.pallas import tpu as pltpu

E = 16
H = 1024
FFN = 2048
TOP_K = 2
T = 128


def _moe_body(x_ref, wr_ref, w1_ref, b1_ref, w2_ref, b2_ref,
              out_ref, probs_ref, topk_ref, c_ref):
    e = pl.program_id(0)

    @pl.when(e == 0)
    def _router():
        xx = x_ref[...]
        g = jax.lax.dot_general(
            xx, wr_ref[...],
            dimension_numbers=(((1,), (1,)), ((), ())),
            preferred_element_type=jnp.float32)
        m = jnp.max(g, axis=-1, keepdims=True)
        ex = jnp.exp(g - m)
        p = ex / jnp.sum(ex, axis=-1, keepdims=True)
        probs_ref[...] = p
        cols = jax.lax.broadcasted_iota(jnp.int32, (T, E), 1)
        i1 = jnp.argmax(p, axis=-1)
        p1 = jnp.max(p, axis=-1)
        pm = jnp.where(cols == i1[:, None], -1.0, p)
        i2 = jnp.argmax(pm, axis=-1)
        p2 = jnp.max(pm, axis=-1)
        s = p1 + p2 + 1e-9
        p1n = (p1 / s)[:, None]
        p2n = (p2 / s)[:, None]
        topk_ref[...] = jnp.concatenate([i1[:, None], i2[:, None]], axis=1)
        c_ref[...] = (jnp.where(cols == i1[:, None], p1n, 0.0)
                      + jnp.where(cols == i2[:, None], p2n, 0.0))

    cols = jax.lax.broadcasted_iota(jnp.int32, (T, E), 1)
    coef = jnp.sum(jnp.where(cols == e, c_ref[...], 0.0), axis=1,
                   keepdims=True)

    h = jax.lax.dot_general(
        x_ref[...], w1_ref[0], dimension_numbers=(((1,), (1,)), ((), ())),
        preferred_element_type=jnp.float32)
    h = h + b1_ref[0]
    h = 0.5 * h * (1.0 + jax.lax.erf(h * 0.7071067811865476))
    o = jax.lax.dot_general(
        h, w2_ref[0], dimension_numbers=(((1,), (1,)), ((), ())),
        preferred_element_type=jnp.float32)
    contrib = coef * (o + b2_ref[0])

    @pl.when(e == 0)
    def _first():
        out_ref[...] = contrib

    @pl.when(e > 0)
    def _rest():
        out_ref[...] += contrib


@jax.jit
def kernel(x, Wr, W1, b1, W2, b2):
    B, S, _ = x.shape
    xf = x.reshape(T, H)

    out, probs, topk = pl.pallas_call(
        _moe_body,
        grid=(E,),
        in_specs=[
            pl.BlockSpec((T, H), lambda e: (0, 0)),
            pl.BlockSpec((E, H), lambda e: (0, 0)),
            pl.BlockSpec((1, FFN, H), lambda e: (e, 0, 0)),
            pl.BlockSpec((1, 1, FFN), lambda e: (e, 0, 0)),
            pl.BlockSpec((1, H, FFN), lambda e: (e, 0, 0)),
            pl.BlockSpec((1, 1, H), lambda e: (e, 0, 0)),
        ],
        out_specs=[
            pl.BlockSpec((T, H), lambda e: (0, 0)),
            pl.BlockSpec((T, E), lambda e: (0, 0)),
            pl.BlockSpec((T, TOP_K), lambda e: (0, 0)),
        ],
        out_shape=[
            jax.ShapeDtypeStruct((T, H), jnp.float32),
            jax.ShapeDtypeStruct((T, E), jnp.float32),
            jax.ShapeDtypeStruct((T, TOP_K), jnp.int32),
        ],
        scratch_shapes=[pltpu.VMEM((T, E), jnp.float32)],
        compiler_params=pltpu.CompilerParams(
            dimension_semantics=("arbitrary",),
        ),
    )(xf, Wr, W1, b1.reshape(E, 1, FFN), W2, b2.reshape(E, 1, H))

    return (out.reshape(B, S, H), probs.reshape(B, S, E),
            topk.reshape(B, S, TOP_K))
